# baseline (speedup 1.0000x reference)
import functools

import jax
import jax.numpy as jnp
from jax import lax
from jax.experimental import pallas as pl
from jax.experimental.pallas import tpu as pltpu
from jax.experimental.pallas import tpu_sc as plsc

N = 10000
E = 160000
D = 256
HALF = 128
ALPHA = 0.2

L = 16
NS = 16
NC = 2

CH_A = 313
EPT_A = CH_A * L
EA = 32 * EPT_A
BE = 96
EPT_R = (E + N) // NS
EPT_C = 10752
E2 = NS * EPT_C
NCH_C = EPT_C // L
NP = 10240
NPT_A = NP // NS
RSTEP = 2560
DROW = 2560
RPAD = 2688
FCH = 40
ZCH = RPAD // NS
PSZ = EPT_C + 192

_mesh = plsc.VectorSubcoreMesh(core_axis_name="c", subcore_axis_name="s")


def _rsqrt16(v):
    bits = plsc.bitcast(v, jnp.int32)
    y = plsc.bitcast(jnp.int32(0x5F3759DF) - lax.shift_right_arithmetic(bits, 1),
                     jnp.float32)
    for _ in range(3):
        y = y * (1.5 - 0.5 * v * y * y)
    return y


@functools.partial(
    pl.kernel,
    out_type=jax.ShapeDtypeStruct((NC * NP,), jnp.float32),
    mesh=_mesh,
    scratch_types=[
        pltpu.VMEM((EPT_A,), jnp.int32),
        pltpu.VMEM((EPT_A,), jnp.float32),
        pltpu.VMEM((NP,), jnp.float32),
        pltpu.VMEM((NPT_A,), jnp.float32),
        pltpu.VMEM((NPT_A,), jnp.float32),
        pltpu.VMEM_SHARED((4 * NP,), jnp.float32),
    ],
    compiler_params=pltpu.CompilerParams(needs_layout_passes=False),
)
def _deg_call(col_hbm, ew_hbm, deg_out, colv, ewv, dloc, dsum, dtmp, dsh):
    c = lax.axis_index("c")
    s = lax.axis_index("s")
    wid = c * NS + s
    pltpu.sync_copy(col_hbm.at[pl.ds(wid * EPT_A, EPT_A)], colv)
    pltpu.sync_copy(ew_hbm.at[pl.ds(wid * EPT_A, EPT_A)], ewv)

    def zero_body(i, carry):
        dloc[pl.ds(i * L, L)] = jnp.zeros((L,), jnp.float32)
        return carry

    lax.fori_loop(0, NP // L, zero_body, 0)

    def acc_body(i, carry):
        cid = colv[pl.ds(i * L, L)]
        ew16 = ewv[pl.ds(i * L, L)]
        plsc.addupdate_scatter(dloc, [cid], ew16)
        return carry

    lax.fori_loop(0, CH_A, acc_body, 0)

    nbase = s * NPT_A

    def zs_body(i, carry):
        dsum[pl.ds(i * L, L)] = jnp.zeros((L,), jnp.float32)
        return carry

    lax.fori_loop(0, NPT_A // L, zs_body, 0)

    for w in range(4):

        @pl.when(s // 4 == w)
        def _():
            pltpu.sync_copy(dloc, dsh.at[pl.ds((s % 4) * NP, NP)])

        plsc.subcore_barrier()
        for k in range(4):
            pltpu.sync_copy(dsh.at[pl.ds(k * NP + nbase, NPT_A)], dtmp)

            def add_body(i, carry):
                sl = pl.ds(i * L, L)
                dsum[sl] = dsum[sl] + dtmp[sl]
                return carry

            lax.fori_loop(0, NPT_A // L, add_body, 0)
        plsc.subcore_barrier()

    pltpu.sync_copy(dsum, deg_out.at[pl.ds(c * NP + nbase, NPT_A)])


def _mm_body(x_ref, w_ref, g_ref):
    h = jnp.dot(x_ref[...], w_ref[...], preferred_element_type=jnp.float32)
    h = h * (1.0 - ALPHA)
    g_ref[0] = h[:, :HALF]
    g_ref[1] = h[:, HALF:]


def _mm_call(x, w):
    return pl.pallas_call(
        _mm_body,
        grid=(10,),
        in_specs=[
            pl.BlockSpec((N // 10, D), lambda i: (i, 0)),
            pl.BlockSpec((D, D), lambda i: (0, 0)),
        ],
        out_specs=pl.BlockSpec((2, N // 10, HALF), lambda i: (0, i, 0)),
        out_shape=jax.ShapeDtypeStruct((2, N, HALF), jnp.float32),
    )(x, w)


@functools.partial(
    pl.kernel,
    out_type=jax.ShapeDtypeStruct((N, D), jnp.float32),
    mesh=_mesh,
    scratch_types=[
        pltpu.VMEM((EPT_C,), jnp.int32),
        pltpu.VMEM((EPT_C,), jnp.int32),
        pltpu.VMEM((EPT_C,), jnp.float32),
        pltpu.VMEM((PSZ,), jnp.int32),
        pltpu.VMEM((NP // HALF, HALF), jnp.float32),
        pltpu.VMEM((2, BE, HALF), jnp.float32),
        pltpu.VMEM((2, BE, HALF), jnp.float32),
        pltpu.VMEM((2, BE), jnp.int32),
        pltpu.VMEM((2, BE), jnp.int32),
        pltpu.VMEM((HALF,), jnp.float32),
        pltpu.SemaphoreType.DMA,
        pltpu.SemaphoreType.DMA,
        pltpu.SemaphoreType.DMA,
        pltpu.SemaphoreType.DMA,
        pltpu.VMEM_SHARED((RPAD, HALF), jnp.float32),
    ],
    compiler_params=pltpu.CompilerParams(needs_layout_passes=False),
)
def _msg_call(row_hbm, col_hbm, ew_hbm, deg_hbm, g_hbm, x_hbm, b_hbm, out_hbm,
              rowv, colv, ewv, perm, disv, gbuf, sbuf, gidxv, cidxv,
              bbuf, semg0, semg1, semsc0, semsc1, acc):
    fbuf = sbuf.at[0, pl.ds(0, FCH)]
    xbuf = sbuf.at[1, pl.ds(0, FCH)]
    c = lax.axis_index("c")
    s = lax.axis_index("s")
    semg = (semg0, semg1)
    semsc = (semsc0, semsc1)
    pltpu.sync_copy(row_hbm.at[pl.ds(s * EPT_C, EPT_C)], rowv)
    pltpu.sync_copy(col_hbm.at[pl.ds(s * EPT_C, EPT_C)], colv)
    pltpu.sync_copy(ew_hbm.at[pl.ds(s * EPT_C, EPT_C)], ewv)
    DR = NP // HALF
    pltpu.sync_copy(deg_hbm.at[pl.ds(0, DR)], disv)
    pltpu.sync_copy(b_hbm.at[pl.ds(c * HALF, HALF)], bbuf)

    for h in range(2):
        pltpu.sync_copy(deg_hbm.at[pl.ds(DR + h * FCH, FCH)], fbuf)

        def dsum_body(i, carry):
            for cc in range(HALF // L):
                csl = pl.ds(cc * L, L)
                disv[h * FCH + i, csl] = (disv[h * FCH + i, csl]
                                          + fbuf[i, csl])
            return carry

        lax.fori_loop(0, FCH, dsum_body, 0)

    def dis_body(i, carry):
        for cc in range(HALF // L):
            csl = pl.ds(cc * L, L)
            disv[i, csl] = _rsqrt16(disv[i, csl] + 1.0)
        return carry

    lax.fori_loop(0, DR, dis_body, 0)

    goff = c * N
    dummy = g_hbm.at[pl.ds(0, BE)]
    iota16 = lax.iota(jnp.int32, L)
    PADPOS = EPT_C - L

    def _issue_block(blk, slot):
        for cc in range(BE // L):
            pos = perm[pl.ds(blk * BE + cc * L, L)]
            rid = plsc.load_gather(rowv, [pos])
            gidxv[slot, pl.ds(cc * L, L)] = rid + goff
        pltpu.async_copy(g_hbm.at[gidxv.at[slot]], gbuf.at[slot], semg[slot])

    def round_body(r, rcarry):
        lo = r * RSTEP
        hi = lax.min(lo + RSTEP, jnp.int32(N))

        def zf_body(i, carry):
            for cc in range(HALF // L):
                fbuf[i, pl.ds(cc * L, L)] = jnp.zeros((L,), jnp.float32)
            return carry

        lax.fori_loop(0, FCH, zf_body, 0)
        zbase = s * ZCH
        for k in range(ZCH // FCH):
            nrows = FCH if k < ZCH // FCH else 0
            pltpu.sync_copy(fbuf, acc.at[pl.ds(zbase + k * FCH, FCH)])
        pltpu.sync_copy(fbuf.at[pl.ds(0, ZCH - (ZCH // FCH) * FCH)],
                        acc.at[pl.ds(zbase + (ZCH // FCH) * FCH,
                                     ZCH - (ZCH // FCH) * FCH)])
        plsc.subcore_barrier()

        def part_body(i, cnt):
            cid = colv[pl.ds(i * L, L)]
            sel = jnp.logical_and(cid >= lo, cid < hi)
            dst = plsc.cumsum(jnp.where(sel, 1, 0)) - 1 + cnt
            plsc.store_scatter(perm, [dst], iota16 + i * L, mask=sel)
            npop = plsc.all_reduce_population_count(sel)
            return cnt + npop[0]

        cnt = lax.fori_loop(0, NCH_C, part_body, jnp.int32(0))
        for kk in range(2 * BE // L):
            plsc.store_scatter(perm, [cnt + kk * L + iota16],
                               jnp.full((L,), PADPOS, jnp.int32))
        ngrp = lax.div(cnt + (2 * BE - 1), jnp.int32(2 * BE))

        @pl.when(ngrp > 0)
        def _():
            for b in range(2):
                _issue_block(b, b)

        def grp_body(i, carry):
            for b in range(2):
                blk = 2 * i + b
                gslot = b
                sslot = b

                pltpu.make_async_copy(dummy, gbuf.at[gslot],
                                      semg[gslot]).wait()

                @pl.when(i > 0)
                def _():
                    pltpu.make_async_copy(dummy, sbuf.at[sslot],
                                          semsc[sslot]).wait()

                kbase = blk * BE

                def chunk_body(cc, carry2, gslot=gslot, sslot=sslot):
                    msl = pl.ds(cc * L, L)
                    pos = perm[pl.ds(kbase + cc * L, L)]
                    rid = plsc.load_gather(rowv, [pos])
                    cid = plsc.load_gather(colv, [pos])
                    ew = plsc.load_gather(ewv, [pos])
                    nr = lax.shift_right_logical(rid, 7)
                    nl = lax.bitwise_and(rid, 127)
                    dr = plsc.load_gather(disv, [nr, nl])
                    cl = cid - lo
                    sel = jnp.logical_and(cl >= 0, cl < hi - lo)
                    a = jnp.where(sel, ew * dr, 0.0)
                    dummy_row = DROW + lax.bitwise_and(rid, 63)
                    cidxv[sslot, msl] = jnp.where(sel, cl, dummy_row)
                    rbase = cc * L
                    for j in range(L):
                        sv = lax.broadcast(a[j], (L,))
                        row = rbase + j
                        for ff in range(HALF // L):
                            fsl = pl.ds(ff * L, L)
                            sbuf[sslot, row, fsl] = gbuf[gslot, row, fsl] * sv
                    return carry2

                lax.fori_loop(0, BE // L, chunk_body, 0)

                pltpu.async_copy(sbuf.at[sslot], acc.at[cidxv.at[sslot]],
                                 semsc[sslot], add=True)

                @pl.when(i < ngrp - 1)
                def _():
                    _issue_block(blk + 2, gslot)
            return carry

        lax.fori_loop(0, ngrp, grp_body, 0)

        @pl.when(ngrp > 0)
        def _():
            for b in range(2):
                pltpu.make_async_copy(dummy, sbuf.at[b], semsc[b]).wait()
        plsc.subcore_barrier()

        nfc = lax.div(hi - lo, jnp.int32(FCH))
        for k in range(4):
            cid = k * NS + s

            @pl.when(cid < nfc)
            def _(cid=cid):
                r0l = cid * FCH
                r0g = lo + r0l
                pltpu.sync_copy(acc.at[pl.ds(r0l, FCH)], fbuf)
                pltpu.sync_copy(
                    x_hbm.at[pl.ds(r0g, FCH), pl.ds(c * HALF, HALF)], xbuf)

                def fin_body(i, carry):
                    node = r0g + i
                    nr = lax.shift_right_logical(node, 7)
                    nl = lax.bitwise_and(node, 127)
                    dv = plsc.load_gather(
                        disv, [jnp.full((L,), nr, jnp.int32),
                               jnp.full((L,), nl, jnp.int32)])
                    for cc in range(HALF // L):
                        csl = pl.ds(cc * L, L)
                        v = fbuf[i, csl] * dv + (xbuf[i, csl] * ALPHA
                                                 + bbuf[csl] * (1.0 - ALPHA))
                        fbuf[i, csl] = jnp.where(v > 0.0, v, jnp.exp(v) - 1.0)
                    return carry

                lax.fori_loop(0, FCH, fin_body, 0)
                pltpu.sync_copy(
                    fbuf, out_hbm.at[pl.ds(r0g, FCH), pl.ds(c * HALF, HALF)])

        plsc.subcore_barrier()
        return rcarry

    lax.fori_loop(0, 4, round_body, 0)


def kernel(x, edge_index, edge_weight, W, bias):
    f32 = jnp.float32
    i32 = jnp.int32
    row = edge_index[0]
    col = edge_index[1]

    padA_i = jnp.zeros((EA - E,), i32)
    padA_f = jnp.zeros((EA - E,), f32)
    colA = jnp.concatenate([col, padA_i])
    ewA = jnp.concatenate([edge_weight, padA_f])
    deg2 = _deg_call(colA, ewA).reshape(NC * NP // HALF, HALF)

    g3 = _mm_call(x, W)
    g2 = g3.reshape(2 * N, HALF)

    loop_idx = jnp.arange(N, dtype=i32)
    rowR = jnp.concatenate([row, loop_idx]).reshape(NS, EPT_R)
    colR = jnp.concatenate([col, loop_idx]).reshape(NS, EPT_R)
    ewR = jnp.concatenate([edge_weight,
                           jnp.ones((N,), f32)]).reshape(NS, EPT_R)
    padi = jnp.zeros((NS, EPT_C - EPT_R), i32)
    padf = jnp.zeros((NS, EPT_C - EPT_R), f32)
    rowC = jnp.concatenate([rowR, padi], axis=1).reshape(-1)
    colC = jnp.concatenate([colR, padi], axis=1).reshape(-1)
    ewC = jnp.concatenate([ewR, padf], axis=1).reshape(-1)

    return _msg_call(rowC, colC, ewC, deg2, g2, x, bias)


# speedup vs baseline: 1.0825x; 1.0825x over previous
import functools

import jax
import jax.numpy as jnp
from jax import lax
from jax.experimental import pallas as pl
from jax.experimental.pallas import tpu as pltpu
from jax.experimental.pallas import tpu_sc as plsc

N = 10000
E = 160000
D = 256
HALF = 128
ALPHA = 0.2

L = 16
NS = 16
NC = 2

CH_A = 313
EPT_A = CH_A * L
EA = 32 * EPT_A
BE = 64
EPT_R = (E + N) // NS
EPT_C = 10752
E2 = NS * EPT_C
NCH_C = EPT_C // L
NP = 10240
NPT_A = NP // NS
RSTEP = 2560
DROW = 2560
RPAD = 2688
FCH = 40
ZCH = RPAD // NS
PSZ = EPT_C + 128

_mesh = plsc.VectorSubcoreMesh(core_axis_name="c", subcore_axis_name="s")


def _rsqrt16(v):
    bits = plsc.bitcast(v, jnp.int32)
    y = plsc.bitcast(jnp.int32(0x5F3759DF) - lax.shift_right_arithmetic(bits, 1),
                     jnp.float32)
    for _ in range(3):
        y = y * (1.5 - 0.5 * v * y * y)
    return y


@functools.partial(
    pl.kernel,
    out_type=jax.ShapeDtypeStruct((NC * NP,), jnp.float32),
    mesh=_mesh,
    scratch_types=[
        pltpu.VMEM((EPT_A,), jnp.int32),
        pltpu.VMEM((EPT_A,), jnp.float32),
        pltpu.VMEM((NP,), jnp.float32),
        pltpu.VMEM((NPT_A,), jnp.float32),
        pltpu.VMEM((NPT_A,), jnp.float32),
        pltpu.VMEM_SHARED((4 * NP,), jnp.float32),
    ],
    compiler_params=pltpu.CompilerParams(needs_layout_passes=False),
)
def _deg_call(col_hbm, ew_hbm, deg_out, colv, ewv, dloc, dsum, dtmp, dsh):
    c = lax.axis_index("c")
    s = lax.axis_index("s")
    wid = c * NS + s
    pltpu.sync_copy(col_hbm.at[pl.ds(wid * EPT_A, EPT_A)], colv)
    pltpu.sync_copy(ew_hbm.at[pl.ds(wid * EPT_A, EPT_A)], ewv)

    def zero_body(i, carry):
        dloc[pl.ds(i * L, L)] = jnp.zeros((L,), jnp.float32)
        return carry

    lax.fori_loop(0, NP // L, zero_body, 0)

    def acc_body(i, carry):
        cid = colv[pl.ds(i * L, L)]
        ew16 = ewv[pl.ds(i * L, L)]
        plsc.addupdate_scatter(dloc, [cid], ew16)
        return carry

    lax.fori_loop(0, CH_A, acc_body, 0)

    nbase = s * NPT_A

    def zs_body(i, carry):
        dsum[pl.ds(i * L, L)] = jnp.zeros((L,), jnp.float32)
        return carry

    lax.fori_loop(0, NPT_A // L, zs_body, 0)

    for w in range(4):

        @pl.when(s // 4 == w)
        def _():
            pltpu.sync_copy(dloc, dsh.at[pl.ds((s % 4) * NP, NP)])

        plsc.subcore_barrier()
        for k in range(4):
            pltpu.sync_copy(dsh.at[pl.ds(k * NP + nbase, NPT_A)], dtmp)

            def add_body(i, carry):
                sl = pl.ds(i * L, L)
                dsum[sl] = dsum[sl] + dtmp[sl]
                return carry

            lax.fori_loop(0, NPT_A // L, add_body, 0)
        plsc.subcore_barrier()

    pltpu.sync_copy(dsum, deg_out.at[pl.ds(c * NP + nbase, NPT_A)])


def _mm_body(x_ref, w_ref, g_ref):
    h = jnp.dot(x_ref[...], w_ref[...], preferred_element_type=jnp.float32)
    h = h * (1.0 - ALPHA)
    g_ref[0] = h[:, :HALF]
    g_ref[1] = h[:, HALF:]


def _mm_call(x, w):
    return pl.pallas_call(
        _mm_body,
        grid=(10,),
        in_specs=[
            pl.BlockSpec((N // 10, D), lambda i: (i, 0)),
            pl.BlockSpec((D, D), lambda i: (0, 0)),
        ],
        out_specs=pl.BlockSpec((2, N // 10, HALF), lambda i: (0, i, 0)),
        out_shape=jax.ShapeDtypeStruct((2, N, HALF), jnp.float32),
    )(x, w)


@functools.partial(
    pl.kernel,
    out_type=jax.ShapeDtypeStruct((N, D), jnp.float32),
    mesh=_mesh,
    scratch_types=[
        pltpu.VMEM((EPT_C,), jnp.int32),
        pltpu.VMEM((EPT_C,), jnp.int32),
        pltpu.VMEM((EPT_C,), jnp.float32),
        pltpu.VMEM((PSZ,), jnp.int32),
        pltpu.VMEM((NP // HALF, HALF), jnp.float32),
        pltpu.VMEM((2, BE, HALF), jnp.float32),
        pltpu.VMEM((2, BE, HALF), jnp.float32),
        pltpu.VMEM((2, BE), jnp.int32),
        pltpu.VMEM((2, BE), jnp.int32),
        pltpu.VMEM((HALF,), jnp.float32),
        pltpu.SemaphoreType.DMA,
        pltpu.SemaphoreType.DMA,
        pltpu.SemaphoreType.DMA,
        pltpu.SemaphoreType.DMA,
        pltpu.VMEM_SHARED((RPAD, HALF), jnp.float32),
    ],
    compiler_params=pltpu.CompilerParams(needs_layout_passes=False),
)
def _msg_call(row_hbm, col_hbm, ew_hbm, deg_hbm, g_hbm, x_hbm, b_hbm, out_hbm,
              rowv, colv, ewv, perm, disv, gbuf, sbuf, gidxv, cidxv,
              bbuf, semg0, semg1, semsc0, semsc1, acc):
    fbuf = sbuf.at[0, pl.ds(0, FCH)]
    xbuf = sbuf.at[1, pl.ds(0, FCH)]
    c = lax.axis_index("c")
    s = lax.axis_index("s")
    semg = (semg0, semg1)
    semsc = (semsc0, semsc1)
    pltpu.sync_copy(row_hbm.at[pl.ds(s * EPT_C, EPT_C)], rowv)
    pltpu.sync_copy(col_hbm.at[pl.ds(s * EPT_C, EPT_C)], colv)
    pltpu.sync_copy(ew_hbm.at[pl.ds(s * EPT_C, EPT_C)], ewv)
    DR = NP // HALF
    pltpu.sync_copy(deg_hbm.at[pl.ds(0, DR)], disv)
    pltpu.sync_copy(b_hbm.at[pl.ds(c * HALF, HALF)], bbuf)

    for h in range(2):
        pltpu.sync_copy(deg_hbm.at[pl.ds(DR + h * FCH, FCH)], fbuf)

        def dsum_body(i, carry):
            for cc in range(HALF // L):
                csl = pl.ds(cc * L, L)
                disv[h * FCH + i, csl] = (disv[h * FCH + i, csl]
                                          + fbuf[i, csl])
            return carry

        lax.fori_loop(0, FCH, dsum_body, 0)

    def dis_body(i, carry):
        for cc in range(HALF // L):
            csl = pl.ds(cc * L, L)
            disv[i, csl] = _rsqrt16(disv[i, csl] + 1.0)
        return carry

    lax.fori_loop(0, DR, dis_body, 0)

    goff = c * N
    dummy = g_hbm.at[pl.ds(0, BE)]
    iota16 = lax.iota(jnp.int32, L)
    PADPOS = EPT_C - L

    def _issue_block(blk, slot):
        for cc in range(BE // L):
            pos = perm[pl.ds(blk * BE + cc * L, L)]
            rid = plsc.load_gather(rowv, [pos])
            gidxv[slot, pl.ds(cc * L, L)] = rid + goff
        pltpu.async_copy(g_hbm.at[gidxv.at[slot]], gbuf.at[slot], semg[slot])

    def round_body(r, rcarry):
        lo = r * RSTEP
        hi = lax.min(lo + RSTEP, jnp.int32(N))

        def zf_body(i, carry):
            for cc in range(HALF // L):
                fbuf[i, pl.ds(cc * L, L)] = jnp.zeros((L,), jnp.float32)
            return carry

        lax.fori_loop(0, FCH, zf_body, 0)
        zbase = s * ZCH
        for k in range(ZCH // FCH):
            nrows = FCH if k < ZCH // FCH else 0
            pltpu.sync_copy(fbuf, acc.at[pl.ds(zbase + k * FCH, FCH)])
        pltpu.sync_copy(fbuf.at[pl.ds(0, ZCH - (ZCH // FCH) * FCH)],
                        acc.at[pl.ds(zbase + (ZCH // FCH) * FCH,
                                     ZCH - (ZCH // FCH) * FCH)])
        plsc.subcore_barrier()

        def part_body(i, cnt):
            cid = colv[pl.ds(i * L, L)]
            sel = jnp.logical_and(cid >= lo, cid < hi)
            dst = plsc.cumsum(jnp.where(sel, 1, 0)) - 1 + cnt
            plsc.store_scatter(perm, [dst], iota16 + i * L, mask=sel)
            npop = plsc.all_reduce_population_count(sel)
            return cnt + npop[0]

        cnt = lax.fori_loop(0, NCH_C, part_body, jnp.int32(0))
        for kk in range(2 * BE // L):
            plsc.store_scatter(perm, [cnt + kk * L + iota16],
                               jnp.full((L,), PADPOS, jnp.int32))
        ngrp = lax.div(cnt + (2 * BE - 1), jnp.int32(2 * BE))

        @pl.when(ngrp > 0)
        def _():
            for b in range(2):
                _issue_block(b, b)

        def grp_body(i, carry):
            for b in range(2):
                blk = 2 * i + b
                gslot = b
                sslot = b

                pltpu.make_async_copy(dummy, gbuf.at[gslot],
                                      semg[gslot]).wait()

                @pl.when(i > 0)
                def _():
                    pltpu.make_async_copy(dummy, sbuf.at[sslot],
                                          semsc[sslot]).wait()

                kbase = blk * BE

                def chunk_body(cc, carry2, gslot=gslot, sslot=sslot):
                    msl = pl.ds(cc * L, L)
                    pos = perm[pl.ds(kbase + cc * L, L)]
                    rid = plsc.load_gather(rowv, [pos])
                    cid = plsc.load_gather(colv, [pos])
                    ew = plsc.load_gather(ewv, [pos])
                    nr = lax.shift_right_logical(rid, 7)
                    nl = lax.bitwise_and(rid, 127)
                    dr = plsc.load_gather(disv, [nr, nl])
                    cl = cid - lo
                    sel = jnp.logical_and(cl >= 0, cl < hi - lo)
                    a = jnp.where(sel, ew * dr, 0.0)
                    dummy_row = DROW + lax.bitwise_and(rid, 63)
                    cidxv[sslot, msl] = jnp.where(sel, cl, dummy_row)
                    rbase = cc * L
                    for j in range(L):
                        sv = lax.broadcast(a[j], (L,))
                        row = rbase + j
                        for ff in range(HALF // L):
                            fsl = pl.ds(ff * L, L)
                            sbuf[sslot, row, fsl] = gbuf[gslot, row, fsl] * sv
                    return carry2

                lax.fori_loop(0, BE // L, chunk_body, 0)

                pltpu.async_copy(sbuf.at[sslot], acc.at[cidxv.at[sslot]],
                                 semsc[sslot], add=True)

                @pl.when(i < ngrp - 1)
                def _():
                    _issue_block(blk + 2, gslot)
            return carry

        lax.fori_loop(0, ngrp, grp_body, 0)

        @pl.when(ngrp > 0)
        def _():
            for b in range(2):
                pltpu.make_async_copy(dummy, sbuf.at[b], semsc[b]).wait()
        plsc.subcore_barrier()

        nfc = lax.div(hi - lo, jnp.int32(FCH))
        for k in range(4):
            cid = k * NS + s

            @pl.when(cid < nfc)
            def _(cid=cid):
                r0l = cid * FCH
                r0g = lo + r0l
                pltpu.sync_copy(acc.at[pl.ds(r0l, FCH)], fbuf)
                pltpu.sync_copy(
                    x_hbm.at[pl.ds(r0g, FCH), pl.ds(c * HALF, HALF)], xbuf)

                def fin_body(i, carry):
                    node = r0g + i
                    nr = lax.shift_right_logical(node, 7)
                    nl = lax.bitwise_and(node, 127)
                    dv = plsc.load_gather(
                        disv, [jnp.full((L,), nr, jnp.int32),
                               jnp.full((L,), nl, jnp.int32)])
                    for cc in range(HALF // L):
                        csl = pl.ds(cc * L, L)
                        v = fbuf[i, csl] * dv + (xbuf[i, csl] * ALPHA
                                                 + bbuf[csl] * (1.0 - ALPHA))
                        fbuf[i, csl] = jnp.where(v > 0.0, v, jnp.exp(v) - 1.0)
                    return carry

                lax.fori_loop(0, FCH, fin_body, 0)
                pltpu.sync_copy(
                    fbuf, out_hbm.at[pl.ds(r0g, FCH), pl.ds(c * HALF, HALF)])

        plsc.subcore_barrier()
        return rcarry

    lax.fori_loop(0, 4, round_body, 0)


def kernel(x, edge_index, edge_weight, W, bias):
    f32 = jnp.float32
    i32 = jnp.int32
    row = edge_index[0]
    col = edge_index[1]

    padA_i = jnp.zeros((EA - E,), i32)
    padA_f = jnp.zeros((EA - E,), f32)
    colA = jnp.concatenate([col, padA_i])
    ewA = jnp.concatenate([edge_weight, padA_f])
    deg2 = _deg_call(colA, ewA).reshape(NC * NP // HALF, HALF)

    g3 = _mm_call(x, W)
    g2 = g3.reshape(2 * N, HALF)

    loop_idx = jnp.arange(N, dtype=i32)
    rowR = jnp.concatenate([row, loop_idx]).reshape(NS, EPT_R)
    colR = jnp.concatenate([col, loop_idx]).reshape(NS, EPT_R)
    ewR = jnp.concatenate([edge_weight,
                           jnp.ones((N,), f32)]).reshape(NS, EPT_R)
    padi = jnp.zeros((NS, EPT_C - EPT_R), i32)
    padf = jnp.zeros((NS, EPT_C - EPT_R), f32)
    rowC = jnp.concatenate([rowR, padi], axis=1).reshape(-1)
    colC = jnp.concatenate([colR, padi], axis=1).reshape(-1)
    ewC = jnp.concatenate([ewR, padf], axis=1).reshape(-1)

    return _msg_call(rowC, colC, ewC, deg2, g2, x, bias)
